# baseline (speedup 1.0000x reference)
import functools

import jax
import jax.numpy as jnp
from jax import lax
from jax.experimental import pallas as pl
from jax.experimental.pallas import tpu as pltpu
from jax.experimental.pallas import tpu_sc as plsc

B, T, H, W = 4, 16, 64, 64
NCLS = 8192
CO = 16
NG = 9
NROWS = B * T * H
NUM_CORES = 2
NUM_SUBCORES = 16
NW = NUM_CORES * NUM_SUBCORES
ROWS_PER = NROWS // NW
NBUF = 2
TH = T * H


def _sc_body(table_hbm, idx2_hbm, bias_hbm, out_hbm, idx_v, buf_v, acc_v, bias_v, *sems):
    isems, gsems, osems = sems[0:NBUF], sems[NBUF : 2 * NBUF], sems[2 * NBUF :]
    wid = lax.axis_index("c") * NUM_SUBCORES + lax.axis_index("s")
    base = wid * ROWS_PER
    lane = lax.iota(jnp.int32, 16)
    pltpu.sync_copy(bias_hbm, bias_v)
    bvec = bias_v[...]

    def rdecomp(r):
        return r >> 10, (r >> 6) & (T - 1), r & (H - 1)

    def fire_idx(s, r):
        b_, t, h = rdecomp(r)
        for dt in range(3):
            ct = jnp.maximum(t + (dt - 2), 0)
            for dh in range(3):
                ch = jnp.clip(h + (dh - 1), 0, H - 1)
                src = b_ * TH + ct * H + ch
                pltpu.async_copy(idx2_hbm.at[src], idx_v.at[s, dt * 3 + dh], isems[s])

    def fire_gathers(s):
        for g in range(NG):
            pltpu.make_async_copy(idx2_hbm.at[base], idx_v.at[s, g], isems[s]).wait()
        for g in range(NG):
            pltpu.async_copy(
                table_hbm.at[g].at[idx_v.at[s, g]], buf_v.at[s, g], gsems[s]
            )

    def wait_gathers(s):
        for g in range(NG):
            pltpu.make_async_copy(
                table_hbm.at[g].at[idx_v.at[s, g]], buf_v.at[s, g], gsems[s]
            ).wait()

    def wait_out(s):
        pltpu.make_async_copy(
            acc_v.at[s], out_hbm.at[0, :, 0, 0, :], osems[s]
        ).wait()

    def accumulate(s):
        def tree_sum(vals):
            while len(vals) > 1:
                vals = [
                    vals[i] + vals[i + 1] if i + 1 < len(vals) else vals[i]
                    for i in range(0, len(vals), 2)
                ]
            return vals[0]

        def sum_a(w):
            return tree_sum([buf_v[s, g, w, pl.ds(0, 32)] for g in range(NG)])

        def sum_b(w):
            return tree_sum([buf_v[s, g, w, pl.ds(32, 32)] for g in range(NG)])

        def unpk(v):
            return plsc.unpack(
                v, format=plsc.PackFormat.INTERLEAVED,
                preferred_element_type=jnp.float32,
            )

        def emit(wpos, val):
            plsc.store_scatter(acc_v.at[s], [lane, lane * 0 + wpos], val)

        a0, a1 = unpk(sum_a(0))
        b2, _ = unpk(sum_b(1))
        emit(0, (a0 + bvec) + (a1 + b2))

        @pl.loop(1, W - 1, init_carry=a0, unroll=8)
        def _w(wpos, carry):
            a0w, a1w = unpk(sum_a(wpos))
            b2w, _ = unpk(sum_b(wpos + 1))
            emit(wpos, (carry + bvec) + (a1w + b2w))
            return a0w

        a0l, a1l = unpk(sum_a(W - 1))
        b2l, _ = unpk(sum_b(W - 1))
        emit(W - 1, (_w + bvec) + (a1l + b2l))

    for s in range(NBUF):
        fire_idx(s, base + s)
    for s in range(NBUF):
        fire_gathers(s)

    @pl.loop(0, ROWS_PER, step=NBUF)
    def _row(r0):
        for s in range(NBUF):
            rr = r0 + s
            r = base + rr
            wait_gathers(s)

            @pl.when(rr + NBUF < ROWS_PER)
            def _fi():
                fire_idx(s, r + NBUF)

            @pl.when(rr >= NBUF)
            def _wo():
                wait_out(s)

            accumulate(s)
            b_, t, h = rdecomp(r)
            pltpu.async_copy(acc_v.at[s], out_hbm.at[b_, :, t, h, :], osems[s])

            @pl.when(rr + NBUF < ROWS_PER)
            def _fg():
                fire_gathers(s)

    for s in range(NBUF):
        wait_out(s)


_sc_call = functools.partial(
    pl.kernel,
    out_type=jax.ShapeDtypeStruct((B, CO, T, H, W), jnp.float32),
    mesh=plsc.VectorSubcoreMesh(core_axis_name="c", subcore_axis_name="s"),
    scratch_types=[
        pltpu.VMEM((NBUF, NG, W), jnp.int32),
        pltpu.VMEM((NBUF, NG, W, 4 * CO), jnp.bfloat16),
        pltpu.VMEM((NBUF, CO, W), jnp.float32),
        pltpu.VMEM((CO,), jnp.float32),
    ]
    + [pltpu.SemaphoreType.DMA] * (3 * NBUF),
    compiler_params=pltpu.CompilerParams(
        use_tc_tiling_on_sc=False, needs_layout_passes=False
    ),
)(_sc_body)


def kernel(indices, weight, bias):
    wz = jnp.pad(weight.astype(jnp.bfloat16), ((0, 0),) * 4 + ((0, 1),))
    tabp = (
        wz.reshape(CO, NCLS, 3, 3, 2, 2)
        .transpose(2, 3, 1, 4, 0, 5)
        .reshape(NG, NCLS, 4 * CO)
    )
    idx2 = indices.reshape(NROWS, W)
    return _sc_call(tabp, idx2, bias)


# speedup vs baseline: 1.0161x; 1.0161x over previous
import functools

import jax
import jax.numpy as jnp
from jax import lax
from jax.experimental import pallas as pl
from jax.experimental.pallas import tpu as pltpu
from jax.experimental.pallas import tpu_sc as plsc

B, T, H, W = 4, 16, 64, 64
NCLS = 8192
CO = 16
NG = 9
NROWS = B * T * H
NUM_CORES = 2
NUM_SUBCORES = 16
NW = NUM_CORES * NUM_SUBCORES
ROWS_PER = NROWS // NW
NBUF = 2
TH = T * H


def _sc_body(table_hbm, idx2_hbm, bias_hbm, out_hbm, pidx_v, buf_v, acc_v, bias_v, *sems):
    isems, gsems, osems = sems[0:NBUF], sems[NBUF : 2 * NBUF], sems[2 * NBUF :]
    wid = lax.axis_index("c") * NUM_SUBCORES + lax.axis_index("s")
    base = wid * ROWS_PER
    lane = lax.iota(jnp.int32, 16)
    pltpu.sync_copy(bias_hbm, bias_v)
    bvec = bias_v[...]

    def rdecomp(r):
        return r >> 10, (r >> 6) & (T - 1), r & (H - 1)

    for ps in range(2):
        b_, t, _ = rdecomp(base + ps * H)
        for dt in range(3):
            ct = jnp.maximum(t + (dt - 2), 0)
            pltpu.async_copy(
                idx2_hbm.at[pl.ds(b_ * TH + ct * H, H)],
                pidx_v.at[ps, dt],
                isems[0],
            )
    for _ in range(6):
        pltpu.make_async_copy(
            idx2_hbm.at[pl.ds(0, H)], pidx_v.at[0, 0], isems[0]
        ).wait()

    def idx_ref(rr, g):
        h = rr & (H - 1)
        ch = jnp.clip(h + (g % 3 - 1), 0, H - 1)
        return pidx_v.at[(rr >> 6) & 1, g // 3, ch]

    def fire_gathers(s, rr):
        for g in range(NG):
            pltpu.async_copy(
                table_hbm.at[g].at[idx_ref(rr, g)], buf_v.at[s, g], gsems[s]
            )

    def wait_gathers(s, rr):
        for g in range(NG):
            pltpu.make_async_copy(
                table_hbm.at[g].at[idx_ref(rr, g)], buf_v.at[s, g], gsems[s]
            ).wait()

    def wait_out(s):
        pltpu.make_async_copy(
            acc_v.at[s], out_hbm.at[0, :, 0, 0, :], osems[s]
        ).wait()

    def accumulate(s):
        def tree_sum(vals):
            while len(vals) > 1:
                vals = [
                    vals[i] + vals[i + 1] if i + 1 < len(vals) else vals[i]
                    for i in range(0, len(vals), 2)
                ]
            return vals[0]

        def sum_a(w):
            return tree_sum([buf_v[s, g, w, pl.ds(0, 32)] for g in range(NG)])

        def sum_b(w):
            return tree_sum([buf_v[s, g, w, pl.ds(32, 32)] for g in range(NG)])

        def unpk(v):
            return plsc.unpack(
                v, format=plsc.PackFormat.INTERLEAVED,
                preferred_element_type=jnp.float32,
            )

        def emit(wpos, val):
            plsc.store_scatter(acc_v.at[s], [lane, lane * 0 + wpos], val)

        a0, a1 = unpk(sum_a(0))
        b2, _ = unpk(sum_b(1))
        emit(0, (a0 + bvec) + (a1 + b2))

        @pl.loop(1, W - 1, init_carry=a0, unroll=4)
        def _w(wpos, carry):
            a0w, a1w = unpk(sum_a(wpos))
            b2w, _ = unpk(sum_b(wpos + 1))
            emit(wpos, (carry + bvec) + (a1w + b2w))
            return a0w

        a0l, a1l = unpk(sum_a(W - 1))
        b2l, _ = unpk(sum_b(W - 1))
        emit(W - 1, (_w + bvec) + (a1l + b2l))

    for s in range(NBUF):
        fire_gathers(s, s)

    @pl.loop(0, ROWS_PER, step=NBUF)
    def _row(r0):
        for s in range(NBUF):
            rr = r0 + s
            r = base + rr
            wait_gathers(s, rr)

            @pl.when(rr >= NBUF)
            def _wo():
                wait_out(s)

            accumulate(s)
            b_, t, h = rdecomp(r)
            pltpu.async_copy(acc_v.at[s], out_hbm.at[b_, :, t, h, :], osems[s])

            @pl.when(rr + NBUF < ROWS_PER)
            def _fg():
                fire_gathers(s, rr + NBUF)

    for s in range(NBUF):
        wait_out(s)


_sc_call = functools.partial(
    pl.kernel,
    out_type=jax.ShapeDtypeStruct((B, CO, T, H, W), jnp.float32),
    mesh=plsc.VectorSubcoreMesh(core_axis_name="c", subcore_axis_name="s"),
    scratch_types=[
        pltpu.VMEM((2, 3, H, W), jnp.int32),
        pltpu.VMEM((NBUF, NG, W, 4 * CO), jnp.bfloat16),
        pltpu.VMEM((NBUF, CO, W), jnp.float32),
        pltpu.VMEM((CO,), jnp.float32),
    ]
    + [pltpu.SemaphoreType.DMA] * (3 * NBUF),
    compiler_params=pltpu.CompilerParams(
        use_tc_tiling_on_sc=False, needs_layout_passes=False
    ),
)(_sc_body)


def kernel(indices, weight, bias):
    wz = jnp.pad(weight.astype(jnp.bfloat16), ((0, 0),) * 4 + ((0, 1),))
    tabp = (
        wz.reshape(CO, NCLS, 3, 3, 2, 2)
        .transpose(2, 3, 1, 4, 0, 5)
        .reshape(NG, NCLS, 4 * CO)
    )
    idx2 = indices.reshape(NROWS, W)
    return _sc_call(tabp, idx2, bias)
